# baseline (speedup 1.0000x reference)
import functools

import jax
import jax.numpy as jnp
from jax import lax
from jax.experimental import pallas as pl
from jax.experimental.pallas import tpu as pltpu
from jax.experimental.pallas import tpu_sc as plsc

N_TOKENS = 16384
EMBED_DIM = 64
_LANES = 16
_DB = EMBED_DIM // _LANES
_IDX_CHUNK = 128


def _hsum(x):
    lanes = lax.iota(jnp.int32, _LANES)
    for k in (8, 4, 2, 1):
        perm = lax.bitwise_xor(lanes, jnp.int32(k))
        x = x + x.at[perm].get(mode="promise_in_bounds")
    return x


def _rsqrt_newton(x):
    i = lax.bitcast_convert_type(x, jnp.int32)
    i = jnp.int32(0x5F3759DF) - lax.shift_right_arithmetic(i, 1)
    y = lax.bitcast_convert_type(i, jnp.float32)
    for _ in range(3):
        y = y * (1.5 - 0.5 * x * y * y)
    return y


@functools.lru_cache(maxsize=None)
def _build_sc_kernel():
    info = plsc.get_sparse_core_info()
    nc, ns = info.num_cores, info.num_subcores
    nw = nc * ns
    bpw = N_TOKENS // nw
    n_chunks = bpw // _IDX_CHUNK
    mesh = plsc.VectorSubcoreMesh(core_axis_name="c", subcore_axis_name="s")

    @functools.partial(
        pl.kernel,
        mesh=mesh,
        compiler_params=pltpu.CompilerParams(use_tc_tiling_on_sc=False),
        out_type=jax.ShapeDtypeStruct((N_TOKENS, EMBED_DIM), jnp.float32),
        scratch_types=[
            pltpu.VMEM((n_chunks, _IDX_CHUNK), jnp.int32),
            pltpu.VMEM((n_chunks, _IDX_CHUNK), jnp.int32),
            pltpu.VMEM((n_chunks, _IDX_CHUNK), jnp.int32),
            pltpu.VMEM((bpw // _LANES, _LANES), jnp.int32),
            pltpu.VMEM((bpw // _LANES, _LANES), jnp.float32),
            pltpu.VMEM((bpw, EMBED_DIM), jnp.float32),
            pltpu.VMEM((bpw, EMBED_DIM), jnp.float32),
            pltpu.VMEM((bpw, EMBED_DIM), jnp.float32),
            pltpu.VMEM((5, EMBED_DIM), jnp.float32),
            pltpu.VMEM((8, EMBED_DIM), jnp.float32),
            pltpu.SemaphoreType.DMA,
        ],
    )
    def sc_kernel(types_hbm, tvals_hbm, nidx_hbm, i1_hbm, i2_hbm,
                  te_hbm, par_hbm, ntab_hbm, t1_hbm, t2_hbm, out_hbm,
                  nidx_v, i1_v, i2_v, types_v, tvals_v,
                  rows_n, rows_1, rows_2, te_v, par_v, sem):
        wid = lax.axis_index("s") * nc + lax.axis_index("c")
        base = wid * bpw
        cbase = wid * n_chunks

        pltpu.sync_copy(nidx_hbm.at[pl.ds(cbase, n_chunks)], nidx_v)
        pltpu.sync_copy(i1_hbm.at[pl.ds(cbase, n_chunks)], i1_v)
        pltpu.sync_copy(i2_hbm.at[pl.ds(cbase, n_chunks)], i2_v)
        gbase = wid * (bpw // _LANES)
        pltpu.sync_copy(types_hbm.at[pl.ds(gbase, bpw // _LANES)], types_v)
        pltpu.sync_copy(tvals_hbm.at[pl.ds(gbase, bpw // _LANES)], tvals_v)
        pltpu.sync_copy(te_hbm, te_v)
        pltpu.sync_copy(par_hbm, par_v)

        handles = []
        for j in range(n_chunks):
            dst = pl.ds(j * _IDX_CHUNK, _IDX_CHUNK)
            handles.append(pltpu.async_copy(ntab_hbm.at[nidx_v.at[j]], rows_n.at[dst], sem))
            handles.append(pltpu.async_copy(t1_hbm.at[i1_v.at[j]], rows_1.at[dst], sem))
            handles.append(pltpu.async_copy(t2_hbm.at[i2_v.at[j]], rows_2.at[dst], sem))
        for h in handles:
            h.wait()

        vW2 = [par_v[0, pl.ds(db * _LANES, _LANES)] for db in range(_DB)]
        vb2 = [par_v[1, pl.ds(db * _LANES, _LANES)] for db in range(_DB)]
        cw2 = [par_v[2, pl.ds(db * _LANES, _LANES)] for db in range(_DB)]
        cw3 = [par_v[3, pl.ds(db * _LANES, _LANES)] for db in range(_DB)]
        cw4 = [par_v[4, pl.ds(db * _LANES, _LANES)] for db in range(_DB)]
        gam = [par_v[5, pl.ds(db * _LANES, _LANES)] for db in range(_DB)]
        bet = [par_v[6, pl.ds(db * _LANES, _LANES)] for db in range(_DB)]

        def body(g, carry):
            ty16 = types_v[g]
            tv16 = tvals_v[g]
            for l in range(_LANES):
                t = g * _LANES + l
                tvb = jnp.full((_LANES,), tv16[l])
                ty_s = ty16[l]
                accs = []
                for db in range(_DB):
                    sl = pl.ds(db * _LANES, _LANES)
                    te = te_v[ty_s, sl]
                    acc = (te + tvb * vW2[db] + vb2[db]
                           + rows_n[t, sl] * cw2[db]
                           + rows_1[t, sl] * cw3[db]
                           + rows_2[t, sl] * cw4[db])
                    accs.append(acc)
                s = (accs[0] + accs[1]) + (accs[2] + accs[3])
                mu = _hsum(s) * (1.0 / EMBED_DIM)
                d = [accs[db] - mu for db in range(_DB)]
                sq = (d[0] * d[0] + d[1] * d[1]) + (d[2] * d[2] + d[3] * d[3])
                var = _hsum(sq) * (1.0 / EMBED_DIM)
                inv = _rsqrt_newton(var + 1e-5)
                for db in range(_DB):
                    rows_n[t, pl.ds(db * _LANES, _LANES)] = d[db] * inv * gam[db] + bet[db]
            return carry

        lax.fori_loop(0, bpw // _LANES, body, jnp.int32(0))
        pltpu.sync_copy(rows_n, out_hbm.at[pl.ds(base, bpw)])

    return sc_kernel, n_chunks


def kernel(token_types, token_values, node_indices, input1_indices, input2_indices,
           token_emb, value_W, value_b, node_idx_emb, input1_emb, input2_emb,
           combination_weights, ln_gamma, ln_beta):
    sc_kernel, n_chunks = _build_sc_kernel()
    cw = combination_weights
    te_w = token_emb * cw[0][None, :]
    vW2 = value_W[:, 0] * cw[1]
    vb2 = value_b * cw[1]
    params = jnp.concatenate([
        jnp.stack([vW2, vb2, cw[2], cw[3], cw[4], ln_gamma, ln_beta]),
        jnp.zeros((1, EMBED_DIM), jnp.float32)], axis=0)
    tvals = token_values[:, 0].reshape(-1, _LANES)
    nidx = node_indices.reshape(-1, _IDX_CHUNK).astype(jnp.int32)
    i1 = input1_indices.reshape(-1, _IDX_CHUNK).astype(jnp.int32)
    i2 = input2_indices.reshape(-1, _IDX_CHUNK).astype(jnp.int32)
    ttypes = token_types.astype(jnp.int32).reshape(-1, _LANES)
    return sc_kernel(ttypes, tvals, nidx, i1, i2,
                     te_w, params, node_idx_emb, input1_emb, input2_emb)
